# baseline (speedup 1.0000x reference)
import functools

import jax
import jax.numpy as jnp
from jax import lax
from jax.experimental import pallas as pl
from jax.experimental.pallas import tpu as pltpu
from jax.experimental.pallas import tpu_sc as plsc

N = 10000
D = 128
NC = 2
NS = 16
L = 16
CH = 64

E1 = N + 160000
E2 = 160000


G = 6
GW = G * 2 * CH


def _chunks_per_tile(e):
    c = -(-e // (NS * CH))
    return c + (-c % 3)


C1 = _chunks_per_tile(E1)
C2 = _chunks_per_tile(E2)
E1P = NS * C1 * CH
E2P = NS * C2 * CH
RPT = 624
TAIL = N - NS * RPT

NP = 10240
RB = 1024
NRB = NP // RB


def _proj_body(feat_ref, wb_ref, ab_ref, wu_ref, au_ref, h1_ref, h2_ref, sc_ref):
    i = pl.program_id(0)
    fb = feat_ref[0]
    fu = feat_ref[1]
    wb = wb_ref[...]
    wu = wu_ref[...]
    hu1 = jnp.dot(fu, wu, preferred_element_type=jnp.float32)
    hu2 = jnp.dot(fb, wu, preferred_element_type=jnp.float32)
    hb1 = jnp.dot(fb, wb, preferred_element_type=jnp.float32)
    hb2 = jnp.dot(fu, wb, preferred_element_type=jnp.float32)
    h1_ref[0] = hu1
    h1_ref[1] = hb1
    h2_ref[0] = hu2
    h2_ref[1] = hb2
    au = au_ref[...]
    ab = ab_ref[...]
    col = pl.ds(i * RB, RB)
    sc_ref[0, 0, col] = jnp.dot(hu1, au[:D], preferred_element_type=jnp.float32)[:, 0]
    sc_ref[0, 1, col] = jnp.dot(hu1, au[D:], preferred_element_type=jnp.float32)[:, 0]
    sc_ref[0, 2, col] = jnp.dot(hu2, au[:D], preferred_element_type=jnp.float32)[:, 0]
    sc_ref[0, 3, col] = jnp.dot(hu2, au[D:], preferred_element_type=jnp.float32)[:, 0]
    sc_ref[1, 0, col] = jnp.dot(hb1, ab[:D], preferred_element_type=jnp.float32)[:, 0]
    sc_ref[1, 1, col] = jnp.dot(hb1, ab[D:], preferred_element_type=jnp.float32)[:, 0]
    sc_ref[1, 2, col] = jnp.dot(hb2, ab[:D], preferred_element_type=jnp.float32)[:, 0]
    sc_ref[1, 3, col] = jnp.dot(hb2, ab[D:], preferred_element_type=jnp.float32)[:, 0]


def _fin_body(acc_ref, den_ref, xb_ref, xu_ref):
    den_u = den_ref[0] + 1e-16
    den_b = den_ref[1] + 1e-16
    xu_ref[...] = acc_ref[0] / den_u[:, None]
    xb_ref[...] = acc_ref[1] / den_b[:, None]


def _edge_body(h1f, h2f, scal, e1, e2, zrows,
               acc_out, den_out,
               acc_sp, den_sp, tv, sv, idxf, rbuf, cbuf2, exv, rows, zbuf,
               si_sem, sd_sem, sg_sem, ss_sem):
    c = lax.axis_index("c")
    s = lax.axis_index("s")
    coff = c * NP

    def _zero_zbuf(i, _):
        zbuf[pl.ds(i * L, L)] = jnp.zeros((L,), jnp.float32)
        return 0

    lax.fori_loop(0, RPT // L, _zero_zbuf, 0, unroll=4)
    pltpu.sync_copy(zbuf, den_sp.at[pl.ds(s * RPT, RPT)])
    pltpu.sync_copy(zrows.at[pl.ds(s * RPT, RPT)], acc_sp.at[pl.ds(s * RPT, RPT)])

    @pl.when(s == 0)
    def _zero_tail():
        pltpu.sync_copy(zbuf.at[pl.ds(0, TAIL)],
                        den_sp.at[pl.ds(NS * RPT, TAIL)])
        pltpu.sync_copy(zrows.at[pl.ds(NS * RPT, TAIL)],
                        acc_sp.at[pl.ds(NS * RPT, TAIL)])

    plsc.subcore_barrier()

    def process(edges, set_idx, hf, n_chunks):
        pltpu.sync_copy(scal.at[pl.ds((c * 4 + 2 * set_idx) * NP, N)], tv)
        pltpu.sync_copy(scal.at[pl.ds((c * 4 + 2 * set_idx + 1) * NP, N)],
                        sv.at[pl.ds(0, N)])
        sv[pl.ds(N, L)] = jnp.full((L,), -1e30, jnp.float32)

        tile_base = s * n_chunks * 2 * CH

        def prep(jj, q):
            base = tile_base + jj * 2 * CH
            pltpu.sync_copy(edges.at[pl.ds(base, 2 * CH)], idxf.at[q])
            for g in range(CH // L):
                r16 = idxf[q, pl.ds(g * L, L)]
                c16 = idxf[q, pl.ds(CH + g * L, L)]
                t = plsc.load_gather(tv, [r16])
                sval = plsc.load_gather(sv, [c16])
                e = t + sval
                e = jnp.maximum(e, 0.2 * e)
                ex = jnp.exp(e)
                exv[q, pl.ds(g * L, L)] = ex
                rbuf[q, pl.ds(g * L, L)] = r16
                cbuf2[q, pl.ds(g * L, L)] = c16 + coff
            pltpu.sync_copy(exv.at[q], den_sp.at[rbuf.at[q]], add=True)
            pltpu.async_copy(hf.at[cbuf2.at[q]], rows.at[q], sg_sem.at[q])

        def finish(q):
            pltpu.make_async_copy(hf.at[cbuf2.at[q]], rows.at[q],
                                  sg_sem.at[q]).wait()

            @plsc.parallel_loop(0, CH, step=1, unroll=4)
            def rowb(k):
                exb = plsc.load_gather(exv.at[q], [jnp.zeros((L,), jnp.int32) + k])
                for f in range(D // L):
                    rows[q, k, pl.ds(f * L, L)] = rows[q, k, pl.ds(f * L, L)] * exb
            pltpu.async_copy(rows.at[q], acc_sp.at[rbuf.at[q]], ss_sem.at[q],
                             add=True)

        def wait_sc(q):
            pltpu.make_async_copy(rows.at[q], acc_sp.at[rbuf.at[q]],
                                  ss_sem.at[q]).wait()

        prep(0, 0)
        prep(1, 1)
        ntri = n_chunks // 3

        def tri(p, _):
            j = p * 3
            finish(0)

            @pl.when(p > 0)
            def _w2():
                wait_sc(2)

            prep(j + 2, 2)
            finish(1)
            wait_sc(0)

            @pl.when(p + 1 < ntri)
            def _p0():
                prep(j + 3, 0)

            finish(2)
            wait_sc(1)

            @pl.when(p + 1 < ntri)
            def _p1():
                prep(j + 4, 1)

            return 0

        lax.fori_loop(0, ntri, tri, 0)
        wait_sc(2)

    process(e1, 0, h1f, C1)
    process(e2, 1, h2f, C2)

    plsc.subcore_barrier()
    pltpu.sync_copy(den_sp.at[pl.ds(s * RPT, RPT)], zbuf)
    pltpu.sync_copy(zbuf, den_out.at[pl.ds(c * N + s * RPT, RPT)])
    pltpu.sync_copy(acc_sp.at[pl.ds(s * RPT, RPT)],
                    acc_out.at[c, pl.ds(s * RPT, RPT)])

    @pl.when(s == 0)
    def _copy_tail():
        pltpu.sync_copy(den_sp.at[pl.ds(NS * RPT, TAIL)],
                        zbuf.at[pl.ds(0, TAIL)])
        pltpu.sync_copy(zbuf.at[pl.ds(0, TAIL)],
                        den_out.at[pl.ds(c * N + NS * RPT, TAIL)])
        pltpu.sync_copy(acc_sp.at[pl.ds(NS * RPT, TAIL)],
                        acc_out.at[c, pl.ds(NS * RPT, TAIL)])


_edge_kernel = functools.partial(
    pl.kernel,
    out_type=[
        jax.ShapeDtypeStruct((NC, N, D), jnp.float32),
        jax.ShapeDtypeStruct((NC * N,), jnp.float32),
    ],
    mesh=plsc.VectorSubcoreMesh(
        core_axis_name="c", subcore_axis_name="s", num_cores=NC,
        num_subcores=NS),
    compiler_params=pltpu.CompilerParams(needs_layout_passes=False),
    scratch_types=[
        pltpu.VMEM_SHARED((N, D), jnp.float32),
        pltpu.VMEM_SHARED((N,), jnp.float32),
        pltpu.VMEM((N,), jnp.float32),
        pltpu.VMEM((N + L,), jnp.float32),
        pltpu.VMEM((3, 2 * CH), jnp.int32),
        pltpu.VMEM((3, CH), jnp.int32),
        pltpu.VMEM((3, CH), jnp.int32),
        pltpu.VMEM((3, CH), jnp.float32),
        pltpu.VMEM((3, CH, D), jnp.float32),
        pltpu.VMEM((RPT,), jnp.float32),
        pltpu.SemaphoreType.DMA((2,)),
        pltpu.SemaphoreType.DMA((3,)),
        pltpu.SemaphoreType.DMA((3,)),
        pltpu.SemaphoreType.DMA((3,)),
    ],
)(_edge_body)


def _pad_chunked(r, c, ep):
    r = jnp.pad(r.astype(jnp.int32), (0, ep - r.shape[0]))
    c = jnp.pad(c.astype(jnp.int32), (0, ep - c.shape[0]),
                constant_values=N)
    nch = ep // CH
    return jnp.stack([r.reshape(nch, CH), c.reshape(nch, CH)],
                     axis=1).reshape(-1)


@jax.jit
def kernel(nodes, adj_pos, adj_neg, feat_table, W_bal, a_bal, W_unbal, a_unbal):
    h1f, h2f, scal = pl.pallas_call(
        _proj_body,
        grid=(NRB,),
        in_specs=[
            pl.BlockSpec((2, RB, D), lambda i: (0, i, 0)),
            pl.BlockSpec((D, D), lambda i: (0, 0)),
            pl.BlockSpec((2 * D, 1), lambda i: (0, 0)),
            pl.BlockSpec((D, D), lambda i: (0, 0)),
            pl.BlockSpec((2 * D, 1), lambda i: (0, 0)),
        ],
        out_specs=[
            pl.BlockSpec((NC, RB, D), lambda i: (0, i, 0)),
            pl.BlockSpec((NC, RB, D), lambda i: (0, i, 0)),
            pl.BlockSpec((NC, 4, NP), lambda i: (0, 0, 0)),
        ],
        out_shape=[
            jax.ShapeDtypeStruct((NC, NP, D), jnp.float32),
            jax.ShapeDtypeStruct((NC, NP, D), jnp.float32),
            jax.ShapeDtypeStruct((NC, 4, NP), jnp.float32),
        ],
    )(jnp.pad(feat_table, ((0, 0), (0, NP - N), (0, 0))),
      W_bal, a_bal, W_unbal, a_unbal)

    rows_self = jnp.arange(N, dtype=jnp.int32)
    e1 = _pad_chunked(
        jnp.concatenate([rows_self, adj_pos[0].astype(jnp.int32)]),
        jnp.concatenate([nodes.astype(jnp.int32), adj_pos[1].astype(jnp.int32)]),
        E1P)
    e2 = _pad_chunked(adj_neg[0], adj_neg[1], E2P)

    zrows = jnp.zeros((N, D), jnp.float32)

    acc, den = _edge_kernel(
        h1f.reshape(NC * NP, D), h2f.reshape(NC * NP, D), scal.reshape(-1),
        e1, e2, zrows)
    den = den.reshape(NC, N)

    x_bal, x_unbal = pl.pallas_call(
        _fin_body,
        out_shape=[
            jax.ShapeDtypeStruct((N, D), jnp.float32),
            jax.ShapeDtypeStruct((N, D), jnp.float32),
        ],
    )(acc, den)
    return (x_bal, x_unbal)


# speedup vs baseline: 1.0006x; 1.0006x over previous
import functools

import jax
import jax.numpy as jnp
from jax import lax
from jax.experimental import pallas as pl
from jax.experimental.pallas import tpu as pltpu
from jax.experimental.pallas import tpu_sc as plsc

N = 10000
D = 128
NC = 2
NS = 16
L = 16
CH = 64

E1 = N + 160000
E2 = 160000


G = 6
GW = G * 2 * CH


def _chunks_per_tile(e):
    c = -(-e // (NS * CH))
    return c + (-c % 3)


C1 = _chunks_per_tile(E1)
C2 = _chunks_per_tile(E2)
E1P = NS * C1 * CH
E2P = NS * C2 * CH
RPT = 624
TAIL = N - NS * RPT

NP = 10240
RB = 1024
NRB = NP // RB


def _proj_body(feat_ref, wb_ref, ab_ref, wu_ref, au_ref, h1_ref, h2_ref, sc_ref):
    i = pl.program_id(0)
    fb = feat_ref[0]
    fu = feat_ref[1]
    wb = wb_ref[...]
    wu = wu_ref[...]
    hu1 = jnp.dot(fu, wu, preferred_element_type=jnp.float32)
    hu2 = jnp.dot(fb, wu, preferred_element_type=jnp.float32)
    hb1 = jnp.dot(fb, wb, preferred_element_type=jnp.float32)
    hb2 = jnp.dot(fu, wb, preferred_element_type=jnp.float32)
    h1_ref[0] = hu1
    h1_ref[1] = hb1
    h2_ref[0] = hu2
    h2_ref[1] = hb2
    au = au_ref[...]
    ab = ab_ref[...]
    col = pl.ds(i * RB, RB)
    sc_ref[0, 0, col] = jnp.dot(hu1, au[:D], preferred_element_type=jnp.float32)[:, 0]
    sc_ref[0, 1, col] = jnp.dot(hu1, au[D:], preferred_element_type=jnp.float32)[:, 0]
    sc_ref[0, 2, col] = jnp.dot(hu2, au[:D], preferred_element_type=jnp.float32)[:, 0]
    sc_ref[0, 3, col] = jnp.dot(hu2, au[D:], preferred_element_type=jnp.float32)[:, 0]
    sc_ref[1, 0, col] = jnp.dot(hb1, ab[:D], preferred_element_type=jnp.float32)[:, 0]
    sc_ref[1, 1, col] = jnp.dot(hb1, ab[D:], preferred_element_type=jnp.float32)[:, 0]
    sc_ref[1, 2, col] = jnp.dot(hb2, ab[:D], preferred_element_type=jnp.float32)[:, 0]
    sc_ref[1, 3, col] = jnp.dot(hb2, ab[D:], preferred_element_type=jnp.float32)[:, 0]


def _fin_body(acc_ref, den_ref, xb_ref, xu_ref):
    den_u = den_ref[0] + 1e-16
    den_b = den_ref[1] + 1e-16
    xu_ref[...] = acc_ref[0] / den_u[:, None]
    xb_ref[...] = acc_ref[1] / den_b[:, None]


def _edge_body(h1f, h2f, scal, e1, e2, zrows,
               acc_out, den_out,
               acc_sp, den_sp, tv, sv, idxf, rbuf, cbuf2, exv, rows, zbuf,
               si_sem, sd_sem, sg_sem, ss_sem):
    c = lax.axis_index("c")
    s = lax.axis_index("s")
    coff = c * NP

    def _zero_zbuf(i, _):
        zbuf[pl.ds(i * L, L)] = jnp.zeros((L,), jnp.float32)
        return 0

    lax.fori_loop(0, RPT // L, _zero_zbuf, 0, unroll=4)
    pltpu.sync_copy(zbuf, den_sp.at[pl.ds(s * RPT, RPT)])
    pltpu.sync_copy(zrows.at[pl.ds(s * RPT, RPT)], acc_sp.at[pl.ds(s * RPT, RPT)])

    @pl.when(s == 0)
    def _zero_tail():
        pltpu.sync_copy(zbuf.at[pl.ds(0, TAIL)],
                        den_sp.at[pl.ds(NS * RPT, TAIL)])
        pltpu.sync_copy(zrows.at[pl.ds(NS * RPT, TAIL)],
                        acc_sp.at[pl.ds(NS * RPT, TAIL)])

    plsc.subcore_barrier()

    def process(edges, set_idx, hf, n_chunks):
        pltpu.sync_copy(scal.at[pl.ds((c * 4 + 2 * set_idx) * NP, N)], tv)
        pltpu.sync_copy(scal.at[pl.ds((c * 4 + 2 * set_idx + 1) * NP, N)],
                        sv.at[pl.ds(0, N)])
        sv[pl.ds(N, L)] = jnp.full((L,), -1e30, jnp.float32)

        tile_base = s * n_chunks * 2 * CH

        def prep(jj, q):
            base = tile_base + jj * 2 * CH
            pltpu.sync_copy(edges.at[pl.ds(base, 2 * CH)], idxf.at[q])
            for g in range(CH // L):
                r16 = idxf[q, pl.ds(g * L, L)]
                c16 = idxf[q, pl.ds(CH + g * L, L)]
                t = plsc.load_gather(tv, [r16])
                sval = plsc.load_gather(sv, [c16])
                e = t + sval
                e = jnp.maximum(e, 0.2 * e)
                ex = jnp.exp(e)
                exv[q, pl.ds(g * L, L)] = ex
                rbuf[q, pl.ds(g * L, L)] = r16
                cbuf2[q, pl.ds(g * L, L)] = c16 + coff
            pltpu.sync_copy(exv.at[q], den_sp.at[rbuf.at[q]], add=True)
            pltpu.async_copy(hf.at[cbuf2.at[q]], rows.at[q], sg_sem.at[q])

        def finish(q):
            pltpu.make_async_copy(hf.at[cbuf2.at[q]], rows.at[q],
                                  sg_sem.at[q]).wait()

            @plsc.parallel_loop(0, CH, step=1, unroll=2)
            def rowb(k):
                exb = plsc.load_gather(exv.at[q], [jnp.zeros((L,), jnp.int32) + k])
                for f in range(D // L):
                    rows[q, k, pl.ds(f * L, L)] = rows[q, k, pl.ds(f * L, L)] * exb
            pltpu.async_copy(rows.at[q], acc_sp.at[rbuf.at[q]], ss_sem.at[q],
                             add=True)

        def wait_sc(q):
            pltpu.make_async_copy(rows.at[q], acc_sp.at[rbuf.at[q]],
                                  ss_sem.at[q]).wait()

        prep(0, 0)
        prep(1, 1)
        ntri = n_chunks // 3

        def tri(p, _):
            j = p * 3
            finish(0)

            @pl.when(p > 0)
            def _w2():
                wait_sc(2)

            prep(j + 2, 2)
            finish(1)
            wait_sc(0)

            @pl.when(p + 1 < ntri)
            def _p0():
                prep(j + 3, 0)

            finish(2)
            wait_sc(1)

            @pl.when(p + 1 < ntri)
            def _p1():
                prep(j + 4, 1)

            return 0

        lax.fori_loop(0, ntri, tri, 0)
        wait_sc(2)

    process(e1, 0, h1f, C1)
    process(e2, 1, h2f, C2)

    plsc.subcore_barrier()
    pltpu.sync_copy(den_sp.at[pl.ds(s * RPT, RPT)], zbuf)
    pltpu.sync_copy(zbuf, den_out.at[pl.ds(c * N + s * RPT, RPT)])
    pltpu.sync_copy(acc_sp.at[pl.ds(s * RPT, RPT)],
                    acc_out.at[c, pl.ds(s * RPT, RPT)])

    @pl.when(s == 0)
    def _copy_tail():
        pltpu.sync_copy(den_sp.at[pl.ds(NS * RPT, TAIL)],
                        zbuf.at[pl.ds(0, TAIL)])
        pltpu.sync_copy(zbuf.at[pl.ds(0, TAIL)],
                        den_out.at[pl.ds(c * N + NS * RPT, TAIL)])
        pltpu.sync_copy(acc_sp.at[pl.ds(NS * RPT, TAIL)],
                        acc_out.at[c, pl.ds(NS * RPT, TAIL)])


_edge_kernel = functools.partial(
    pl.kernel,
    out_type=[
        jax.ShapeDtypeStruct((NC, N, D), jnp.float32),
        jax.ShapeDtypeStruct((NC * N,), jnp.float32),
    ],
    mesh=plsc.VectorSubcoreMesh(
        core_axis_name="c", subcore_axis_name="s", num_cores=NC,
        num_subcores=NS),
    compiler_params=pltpu.CompilerParams(needs_layout_passes=False),
    scratch_types=[
        pltpu.VMEM_SHARED((N, D), jnp.float32),
        pltpu.VMEM_SHARED((N,), jnp.float32),
        pltpu.VMEM((N,), jnp.float32),
        pltpu.VMEM((N + L,), jnp.float32),
        pltpu.VMEM((3, 2 * CH), jnp.int32),
        pltpu.VMEM((3, CH), jnp.int32),
        pltpu.VMEM((3, CH), jnp.int32),
        pltpu.VMEM((3, CH), jnp.float32),
        pltpu.VMEM((3, CH, D), jnp.float32),
        pltpu.VMEM((RPT,), jnp.float32),
        pltpu.SemaphoreType.DMA((2,)),
        pltpu.SemaphoreType.DMA((3,)),
        pltpu.SemaphoreType.DMA((3,)),
        pltpu.SemaphoreType.DMA((3,)),
    ],
)(_edge_body)


def _pad_chunked(r, c, ep):
    r = jnp.pad(r.astype(jnp.int32), (0, ep - r.shape[0]))
    c = jnp.pad(c.astype(jnp.int32), (0, ep - c.shape[0]),
                constant_values=N)
    nch = ep // CH
    return jnp.stack([r.reshape(nch, CH), c.reshape(nch, CH)],
                     axis=1).reshape(-1)


@jax.jit
def kernel(nodes, adj_pos, adj_neg, feat_table, W_bal, a_bal, W_unbal, a_unbal):
    h1f, h2f, scal = pl.pallas_call(
        _proj_body,
        grid=(NRB,),
        in_specs=[
            pl.BlockSpec((2, RB, D), lambda i: (0, i, 0)),
            pl.BlockSpec((D, D), lambda i: (0, 0)),
            pl.BlockSpec((2 * D, 1), lambda i: (0, 0)),
            pl.BlockSpec((D, D), lambda i: (0, 0)),
            pl.BlockSpec((2 * D, 1), lambda i: (0, 0)),
        ],
        out_specs=[
            pl.BlockSpec((NC, RB, D), lambda i: (0, i, 0)),
            pl.BlockSpec((NC, RB, D), lambda i: (0, i, 0)),
            pl.BlockSpec((NC, 4, NP), lambda i: (0, 0, 0)),
        ],
        out_shape=[
            jax.ShapeDtypeStruct((NC, NP, D), jnp.float32),
            jax.ShapeDtypeStruct((NC, NP, D), jnp.float32),
            jax.ShapeDtypeStruct((NC, 4, NP), jnp.float32),
        ],
    )(jnp.pad(feat_table, ((0, 0), (0, NP - N), (0, 0))),
      W_bal, a_bal, W_unbal, a_unbal)

    rows_self = jnp.arange(N, dtype=jnp.int32)
    e1 = _pad_chunked(
        jnp.concatenate([rows_self, adj_pos[0].astype(jnp.int32)]),
        jnp.concatenate([nodes.astype(jnp.int32), adj_pos[1].astype(jnp.int32)]),
        E1P)
    e2 = _pad_chunked(adj_neg[0], adj_neg[1], E2P)

    zrows = jnp.zeros((N, D), jnp.float32)

    acc, den = _edge_kernel(
        h1f.reshape(NC * NP, D), h2f.reshape(NC * NP, D), scal.reshape(-1),
        e1, e2, zrows)
    den = den.reshape(NC, N)

    x_bal, x_unbal = pl.pallas_call(
        _fin_body,
        out_shape=[
            jax.ShapeDtypeStruct((N, D), jnp.float32),
            jax.ShapeDtypeStruct((N, D), jnp.float32),
        ],
    )(acc, den)
    return (x_bal, x_unbal)


# speedup vs baseline: 1.0869x; 1.0862x over previous
import functools

import jax
import jax.numpy as jnp
from jax import lax
from jax.experimental import pallas as pl
from jax.experimental.pallas import tpu as pltpu
from jax.experimental.pallas import tpu_sc as plsc

N = 10000
D = 128
NC = 2
NS = 16
L = 16
CH = 64

E1 = N + 160000
E2 = 160000


G = 6
GW = G * 2 * CH


def _chunks_per_tile(e):
    c = -(-e // (NS * CH))
    return c + (-c % 3)


C1 = _chunks_per_tile(E1)
C2 = _chunks_per_tile(E2)
E1P = NS * C1 * CH
E2P = NS * C2 * CH
RPT = 624
TAIL = N - NS * RPT

NP = 10240
RB = 1024
NRB = NP // RB


def _proj_body(feat_ref, wb_ref, ab_ref, wu_ref, au_ref, h1_ref, h2_ref, sc_ref):
    i = pl.program_id(0)
    fb = feat_ref[0]
    fu = feat_ref[1]
    wb = wb_ref[...]
    wu = wu_ref[...]
    hu1 = jnp.dot(fu, wu, preferred_element_type=jnp.float32)
    hu2 = jnp.dot(fb, wu, preferred_element_type=jnp.float32)
    hb1 = jnp.dot(fb, wb, preferred_element_type=jnp.float32)
    hb2 = jnp.dot(fu, wb, preferred_element_type=jnp.float32)
    h1_ref[0] = hu1
    h1_ref[1] = hb1
    h2_ref[0] = hu2
    h2_ref[1] = hb2
    au = au_ref[...]
    ab = ab_ref[...]
    col = pl.ds(i * RB, RB)
    sc_ref[0, 0, col] = jnp.dot(hu1, au[:D], preferred_element_type=jnp.float32)[:, 0]
    sc_ref[0, 1, col] = jnp.dot(hu1, au[D:], preferred_element_type=jnp.float32)[:, 0]
    sc_ref[0, 2, col] = jnp.dot(hu2, au[:D], preferred_element_type=jnp.float32)[:, 0]
    sc_ref[0, 3, col] = jnp.dot(hu2, au[D:], preferred_element_type=jnp.float32)[:, 0]
    sc_ref[1, 0, col] = jnp.dot(hb1, ab[:D], preferred_element_type=jnp.float32)[:, 0]
    sc_ref[1, 1, col] = jnp.dot(hb1, ab[D:], preferred_element_type=jnp.float32)[:, 0]
    sc_ref[1, 2, col] = jnp.dot(hb2, ab[:D], preferred_element_type=jnp.float32)[:, 0]
    sc_ref[1, 3, col] = jnp.dot(hb2, ab[D:], preferred_element_type=jnp.float32)[:, 0]


def _fin_body(acc_ref, den_ref, xb_ref, xu_ref):
    den_u = den_ref[0] + 1e-16
    den_b = den_ref[1] + 1e-16
    xu_ref[...] = acc_ref[0] / den_u[:, None]
    xb_ref[...] = acc_ref[1] / den_b[:, None]


def _edge_body(h1f, h2f, scal, e1, e2, zrows,
               acc_out, den_out,
               acc_sp, den_sp, tv, sv, idxf, rbuf, cbuf2, exv, rows, zbuf,
               si_sem, sd_sem, sg_sem, ss_sem):
    c = lax.axis_index("c")
    s = lax.axis_index("s")
    coff = c * NP

    def _zero_zbuf(i, _):
        zbuf[pl.ds(i * L, L)] = jnp.zeros((L,), jnp.float32)
        return 0

    lax.fori_loop(0, RPT // L, _zero_zbuf, 0, unroll=4)
    pltpu.sync_copy(zbuf, den_sp.at[pl.ds(s * RPT, RPT)])
    pltpu.sync_copy(zrows.at[pl.ds(s * RPT, RPT)], acc_sp.at[pl.ds(s * RPT, RPT)])

    @pl.when(s == 0)
    def _zero_tail():
        pltpu.sync_copy(zbuf.at[pl.ds(0, TAIL)],
                        den_sp.at[pl.ds(NS * RPT, TAIL)])
        pltpu.sync_copy(zrows.at[pl.ds(NS * RPT, TAIL)],
                        acc_sp.at[pl.ds(NS * RPT, TAIL)])

    plsc.subcore_barrier()

    def process(edges, set_idx, hf, n_chunks):
        pltpu.sync_copy(scal.at[pl.ds((c * 4 + 2 * set_idx) * NP, N)], tv)
        pltpu.sync_copy(scal.at[pl.ds((c * 4 + 2 * set_idx + 1) * NP, N)],
                        sv.at[pl.ds(0, N)])
        sv[pl.ds(N, L)] = jnp.full((L,), -1e30, jnp.float32)

        tile_base = s * n_chunks * 2 * CH

        def prep_a(jj, q):
            base = tile_base + jj * 2 * CH
            pltpu.async_copy(edges.at[pl.ds(base, 2 * CH)], idxf.at[q],
                             si_sem.at[q])

        def prep(jj, q):
            base = tile_base + jj * 2 * CH
            pltpu.make_async_copy(edges.at[pl.ds(base, 2 * CH)], idxf.at[q],
                                  si_sem.at[q]).wait()
            for g in range(CH // L):
                r16 = idxf[q, pl.ds(g * L, L)]
                c16 = idxf[q, pl.ds(CH + g * L, L)]
                t = plsc.load_gather(tv, [r16])
                sval = plsc.load_gather(sv, [c16])
                e = t + sval
                e = jnp.maximum(e, 0.2 * e)
                ex = jnp.exp(e)
                exv[q, pl.ds(g * L, L)] = ex
                rbuf[q, pl.ds(g * L, L)] = r16
                cbuf2[q, pl.ds(g * L, L)] = c16 + coff
            pltpu.sync_copy(exv.at[q], den_sp.at[rbuf.at[q]], add=True)
            pltpu.async_copy(hf.at[cbuf2.at[q]], rows.at[q], sg_sem.at[q])

        def finish(q):
            pltpu.make_async_copy(hf.at[cbuf2.at[q]], rows.at[q],
                                  sg_sem.at[q]).wait()

            @plsc.parallel_loop(0, CH, step=1, unroll=2)
            def rowb(k):
                exb = plsc.load_gather(exv.at[q], [jnp.zeros((L,), jnp.int32) + k])
                for f in range(D // L):
                    rows[q, k, pl.ds(f * L, L)] = rows[q, k, pl.ds(f * L, L)] * exb
            pltpu.async_copy(rows.at[q], acc_sp.at[rbuf.at[q]], ss_sem.at[q],
                             add=True)

        def wait_sc(q):
            pltpu.make_async_copy(rows.at[q], acc_sp.at[rbuf.at[q]],
                                  ss_sem.at[q]).wait()

        ntri = n_chunks // 3
        prep_a(0, 0)
        prep_a(1, 1)
        prep_a(2, 2)
        prep(0, 0)
        prep_a(3, 0)
        prep(1, 1)
        prep_a(4, 1)

        def tri(p, _):
            j = p * 3
            finish(0)

            @pl.when(p > 0)
            def _w2():
                wait_sc(2)

            prep(j + 2, 2)

            @pl.when(p + 1 < ntri)
            def _a2():
                prep_a(j + 5, 2)

            finish(1)
            wait_sc(0)

            @pl.when(p + 1 < ntri)
            def _p0():
                prep(j + 3, 0)

            @pl.when(p + 2 < ntri)
            def _a0():
                prep_a(j + 6, 0)

            finish(2)
            wait_sc(1)

            @pl.when(p + 1 < ntri)
            def _p1():
                prep(j + 4, 1)

            @pl.when(p + 2 < ntri)
            def _a1():
                prep_a(j + 7, 1)

            return 0

        lax.fori_loop(0, ntri, tri, 0)
        wait_sc(2)

    process(e1, 0, h1f, C1)
    process(e2, 1, h2f, C2)

    plsc.subcore_barrier()
    pltpu.sync_copy(den_sp.at[pl.ds(s * RPT, RPT)], zbuf)
    pltpu.sync_copy(zbuf, den_out.at[pl.ds(c * N + s * RPT, RPT)])
    pltpu.sync_copy(acc_sp.at[pl.ds(s * RPT, RPT)],
                    acc_out.at[c, pl.ds(s * RPT, RPT)])

    @pl.when(s == 0)
    def _copy_tail():
        pltpu.sync_copy(den_sp.at[pl.ds(NS * RPT, TAIL)],
                        zbuf.at[pl.ds(0, TAIL)])
        pltpu.sync_copy(zbuf.at[pl.ds(0, TAIL)],
                        den_out.at[pl.ds(c * N + NS * RPT, TAIL)])
        pltpu.sync_copy(acc_sp.at[pl.ds(NS * RPT, TAIL)],
                        acc_out.at[c, pl.ds(NS * RPT, TAIL)])


_edge_kernel = functools.partial(
    pl.kernel,
    out_type=[
        jax.ShapeDtypeStruct((NC, N, D), jnp.float32),
        jax.ShapeDtypeStruct((NC * N,), jnp.float32),
    ],
    mesh=plsc.VectorSubcoreMesh(
        core_axis_name="c", subcore_axis_name="s", num_cores=NC,
        num_subcores=NS),
    compiler_params=pltpu.CompilerParams(needs_layout_passes=False),
    scratch_types=[
        pltpu.VMEM_SHARED((N, D), jnp.float32),
        pltpu.VMEM_SHARED((N,), jnp.float32),
        pltpu.VMEM((N,), jnp.float32),
        pltpu.VMEM((N + L,), jnp.float32),
        pltpu.VMEM((3, 2 * CH), jnp.int32),
        pltpu.VMEM((3, CH), jnp.int32),
        pltpu.VMEM((3, CH), jnp.int32),
        pltpu.VMEM((3, CH), jnp.float32),
        pltpu.VMEM((3, CH, D), jnp.float32),
        pltpu.VMEM((RPT,), jnp.float32),
        pltpu.SemaphoreType.DMA((3,)),
        pltpu.SemaphoreType.DMA((3,)),
        pltpu.SemaphoreType.DMA((3,)),
        pltpu.SemaphoreType.DMA((3,)),
    ],
)(_edge_body)


def _pad_chunked(r, c, ep):
    r = jnp.pad(r.astype(jnp.int32), (0, ep - r.shape[0]))
    c = jnp.pad(c.astype(jnp.int32), (0, ep - c.shape[0]),
                constant_values=N)
    nch = ep // CH
    return jnp.stack([r.reshape(nch, CH), c.reshape(nch, CH)],
                     axis=1).reshape(-1)


@jax.jit
def kernel(nodes, adj_pos, adj_neg, feat_table, W_bal, a_bal, W_unbal, a_unbal):
    h1f, h2f, scal = pl.pallas_call(
        _proj_body,
        grid=(NRB,),
        in_specs=[
            pl.BlockSpec((2, RB, D), lambda i: (0, i, 0)),
            pl.BlockSpec((D, D), lambda i: (0, 0)),
            pl.BlockSpec((2 * D, 1), lambda i: (0, 0)),
            pl.BlockSpec((D, D), lambda i: (0, 0)),
            pl.BlockSpec((2 * D, 1), lambda i: (0, 0)),
        ],
        out_specs=[
            pl.BlockSpec((NC, RB, D), lambda i: (0, i, 0)),
            pl.BlockSpec((NC, RB, D), lambda i: (0, i, 0)),
            pl.BlockSpec((NC, 4, NP), lambda i: (0, 0, 0)),
        ],
        out_shape=[
            jax.ShapeDtypeStruct((NC, NP, D), jnp.float32),
            jax.ShapeDtypeStruct((NC, NP, D), jnp.float32),
            jax.ShapeDtypeStruct((NC, 4, NP), jnp.float32),
        ],
    )(jnp.pad(feat_table, ((0, 0), (0, NP - N), (0, 0))),
      W_bal, a_bal, W_unbal, a_unbal)

    rows_self = jnp.arange(N, dtype=jnp.int32)
    e1 = _pad_chunked(
        jnp.concatenate([rows_self, adj_pos[0].astype(jnp.int32)]),
        jnp.concatenate([nodes.astype(jnp.int32), adj_pos[1].astype(jnp.int32)]),
        E1P)
    e2 = _pad_chunked(adj_neg[0], adj_neg[1], E2P)

    zrows = jnp.zeros((N, D), jnp.float32)

    acc, den = _edge_kernel(
        h1f.reshape(NC * NP, D), h2f.reshape(NC * NP, D), scal.reshape(-1),
        e1, e2, zrows)
    den = den.reshape(NC, N)

    x_bal, x_unbal = pl.pallas_call(
        _fin_body,
        out_shape=[
            jax.ShapeDtypeStruct((N, D), jnp.float32),
            jax.ShapeDtypeStruct((N, D), jnp.float32),
        ],
    )(acc, den)
    return (x_bal, x_unbal)


# speedup vs baseline: 1.1045x; 1.0162x over previous
import functools

import jax
import jax.numpy as jnp
from jax import lax
from jax.experimental import pallas as pl
from jax.experimental.pallas import tpu as pltpu
from jax.experimental.pallas import tpu_sc as plsc

N = 10000
D = 128
NC = 2
NS = 16
L = 16
CH = 64

E1 = N + 160000
E2 = 160000


G = 6
GW = G * 2 * CH


def _chunks_per_tile(e):
    c = -(-e // (NS * CH))
    return c + (-c % 3)


C1 = _chunks_per_tile(E1)
C2 = _chunks_per_tile(E2)
E1P = NS * C1 * CH
E2P = NS * C2 * CH
RPT = 624
TAIL = N - NS * RPT

NP = 10240
RB = 1024
NRB = NP // RB


def _proj_body(feat_ref, wb_ref, ab_ref, wu_ref, au_ref, h1_ref, h2_ref, sc_ref):
    i = pl.program_id(0)
    fb = feat_ref[0]
    fu = feat_ref[1]
    wb = wb_ref[...]
    wu = wu_ref[...]
    hu1 = jnp.dot(fu, wu, preferred_element_type=jnp.float32)
    hu2 = jnp.dot(fb, wu, preferred_element_type=jnp.float32)
    hb1 = jnp.dot(fb, wb, preferred_element_type=jnp.float32)
    hb2 = jnp.dot(fu, wb, preferred_element_type=jnp.float32)
    h1_ref[0] = hu1
    h1_ref[1] = hb1
    h2_ref[0] = hu2
    h2_ref[1] = hb2
    au = au_ref[...]
    ab = ab_ref[...]
    col = pl.ds(i * RB, RB)
    sc_ref[0, 0, col] = jnp.dot(hu1, au[:D], preferred_element_type=jnp.float32)[:, 0]
    sc_ref[0, 1, col] = jnp.dot(hu1, au[D:], preferred_element_type=jnp.float32)[:, 0]
    sc_ref[0, 2, col] = jnp.dot(hu2, au[:D], preferred_element_type=jnp.float32)[:, 0]
    sc_ref[0, 3, col] = jnp.dot(hu2, au[D:], preferred_element_type=jnp.float32)[:, 0]
    sc_ref[1, 0, col] = jnp.dot(hb1, ab[:D], preferred_element_type=jnp.float32)[:, 0]
    sc_ref[1, 1, col] = jnp.dot(hb1, ab[D:], preferred_element_type=jnp.float32)[:, 0]
    sc_ref[1, 2, col] = jnp.dot(hb2, ab[:D], preferred_element_type=jnp.float32)[:, 0]
    sc_ref[1, 3, col] = jnp.dot(hb2, ab[D:], preferred_element_type=jnp.float32)[:, 0]


def _fin_body(acc_ref, den_ref, xb_ref, xu_ref):
    den_u = den_ref[0] + 1e-16
    den_b = den_ref[1] + 1e-16
    xu_ref[...] = acc_ref[0] / den_u[:, None]
    xb_ref[...] = acc_ref[1] / den_b[:, None]


def _edge_body(h1f, h2f, scal, e1, e2, zrows,
               acc_out, den_out,
               acc_sp, den_sp, tv, sv, idxf, rbuf, cbuf2, exv, rows, zbuf,
               si_sem, sd_sem, sg_sem, ss_sem):
    c = lax.axis_index("c")
    s = lax.axis_index("s")
    coff = c * NP

    def _zero_zbuf(i, _):
        zbuf[pl.ds(i * L, L)] = jnp.zeros((L,), jnp.float32)
        return 0

    lax.fori_loop(0, RPT // L, _zero_zbuf, 0, unroll=4)
    pltpu.sync_copy(zbuf, den_sp.at[pl.ds(s * RPT, RPT)])
    pltpu.sync_copy(zrows.at[pl.ds(s * RPT, RPT)], acc_sp.at[pl.ds(s * RPT, RPT)])

    @pl.when(s == 0)
    def _zero_tail():
        pltpu.sync_copy(zbuf.at[pl.ds(0, TAIL)],
                        den_sp.at[pl.ds(NS * RPT, TAIL)])
        pltpu.sync_copy(zrows.at[pl.ds(NS * RPT, TAIL)],
                        acc_sp.at[pl.ds(NS * RPT, TAIL)])

    plsc.subcore_barrier()

    def process(edges, set_idx, hf, n_chunks):
        pltpu.sync_copy(scal.at[pl.ds((c * 4 + 2 * set_idx) * NP, N)], tv)
        pltpu.sync_copy(scal.at[pl.ds((c * 4 + 2 * set_idx + 1) * NP, N)],
                        sv.at[pl.ds(0, N)])
        sv[pl.ds(N, L)] = jnp.full((L,), -1e30, jnp.float32)

        tile_base = s * n_chunks * 2 * CH

        def prep_a(jj, q):
            base = tile_base + jj * 2 * CH
            pltpu.async_copy(edges.at[pl.ds(base, 2 * CH)], idxf.at[q],
                             si_sem.at[q])

        def prep(jj, q):
            base = tile_base + jj * 2 * CH
            pltpu.make_async_copy(edges.at[pl.ds(base, 2 * CH)], idxf.at[q],
                                  si_sem.at[q]).wait()
            for g in range(CH // L):
                r16 = idxf[q, pl.ds(g * L, L)]
                c16 = idxf[q, pl.ds(CH + g * L, L)]
                t = plsc.load_gather(tv, [r16])
                sval = plsc.load_gather(sv, [c16])
                e = t + sval
                e = jnp.maximum(e, 0.2 * e)
                ex = jnp.exp(e)
                exv[q, pl.ds(g * L, L)] = ex
                rbuf[q, pl.ds(g * L, L)] = r16
                cbuf2[q, pl.ds(g * L, L)] = c16 + coff
            pltpu.async_copy(hf.at[cbuf2.at[q]], rows.at[q], sg_sem.at[q])
            pltpu.async_copy(exv.at[q], den_sp.at[rbuf.at[q]], sd_sem.at[q],
                             add=True)

        def finish(q):
            pltpu.make_async_copy(hf.at[cbuf2.at[q]], rows.at[q],
                                  sg_sem.at[q]).wait()

            @plsc.parallel_loop(0, CH, step=1, unroll=2)
            def rowb(k):
                exb = plsc.load_gather(exv.at[q], [jnp.zeros((L,), jnp.int32) + k])
                for f in range(D // L):
                    rows[q, k, pl.ds(f * L, L)] = rows[q, k, pl.ds(f * L, L)] * exb
            pltpu.async_copy(rows.at[q], acc_sp.at[rbuf.at[q]], ss_sem.at[q],
                             add=True)

        def wait_sc(q):
            pltpu.make_async_copy(rows.at[q], acc_sp.at[rbuf.at[q]],
                                  ss_sem.at[q]).wait()
            pltpu.make_async_copy(exv.at[q], den_sp.at[rbuf.at[q]],
                                  sd_sem.at[q]).wait()

        ntri = n_chunks // 3
        prep_a(0, 0)
        prep_a(1, 1)
        prep_a(2, 2)
        prep(0, 0)
        prep_a(3, 0)
        prep(1, 1)
        prep_a(4, 1)

        def tri(p, _):
            j = p * 3
            finish(0)

            @pl.when(p > 0)
            def _w2():
                wait_sc(2)

            prep(j + 2, 2)

            @pl.when(p + 1 < ntri)
            def _a2():
                prep_a(j + 5, 2)

            finish(1)
            wait_sc(0)

            @pl.when(p + 1 < ntri)
            def _p0():
                prep(j + 3, 0)

            @pl.when(p + 2 < ntri)
            def _a0():
                prep_a(j + 6, 0)

            finish(2)
            wait_sc(1)

            @pl.when(p + 1 < ntri)
            def _p1():
                prep(j + 4, 1)

            @pl.when(p + 2 < ntri)
            def _a1():
                prep_a(j + 7, 1)

            return 0

        lax.fori_loop(0, ntri, tri, 0)
        wait_sc(2)

    process(e1, 0, h1f, C1)
    process(e2, 1, h2f, C2)

    plsc.subcore_barrier()
    pltpu.sync_copy(den_sp.at[pl.ds(s * RPT, RPT)], zbuf)
    pltpu.sync_copy(zbuf, den_out.at[pl.ds(c * N + s * RPT, RPT)])
    pltpu.sync_copy(acc_sp.at[pl.ds(s * RPT, RPT)],
                    acc_out.at[c, pl.ds(s * RPT, RPT)])

    @pl.when(s == 0)
    def _copy_tail():
        pltpu.sync_copy(den_sp.at[pl.ds(NS * RPT, TAIL)],
                        zbuf.at[pl.ds(0, TAIL)])
        pltpu.sync_copy(zbuf.at[pl.ds(0, TAIL)],
                        den_out.at[pl.ds(c * N + NS * RPT, TAIL)])
        pltpu.sync_copy(acc_sp.at[pl.ds(NS * RPT, TAIL)],
                        acc_out.at[c, pl.ds(NS * RPT, TAIL)])


_edge_kernel = functools.partial(
    pl.kernel,
    out_type=[
        jax.ShapeDtypeStruct((NC, N, D), jnp.float32),
        jax.ShapeDtypeStruct((NC * N,), jnp.float32),
    ],
    mesh=plsc.VectorSubcoreMesh(
        core_axis_name="c", subcore_axis_name="s", num_cores=NC,
        num_subcores=NS),
    compiler_params=pltpu.CompilerParams(needs_layout_passes=False),
    scratch_types=[
        pltpu.VMEM_SHARED((N, D), jnp.float32),
        pltpu.VMEM_SHARED((N,), jnp.float32),
        pltpu.VMEM((N,), jnp.float32),
        pltpu.VMEM((N + L,), jnp.float32),
        pltpu.VMEM((3, 2 * CH), jnp.int32),
        pltpu.VMEM((3, CH), jnp.int32),
        pltpu.VMEM((3, CH), jnp.int32),
        pltpu.VMEM((3, CH), jnp.float32),
        pltpu.VMEM((3, CH, D), jnp.float32),
        pltpu.VMEM((RPT,), jnp.float32),
        pltpu.SemaphoreType.DMA((3,)),
        pltpu.SemaphoreType.DMA((3,)),
        pltpu.SemaphoreType.DMA((3,)),
        pltpu.SemaphoreType.DMA((3,)),
    ],
)(_edge_body)


def _pad_chunked(r, c, ep):
    r = jnp.pad(r.astype(jnp.int32), (0, ep - r.shape[0]))
    c = jnp.pad(c.astype(jnp.int32), (0, ep - c.shape[0]),
                constant_values=N)
    nch = ep // CH
    return jnp.stack([r.reshape(nch, CH), c.reshape(nch, CH)],
                     axis=1).reshape(-1)


@jax.jit
def kernel(nodes, adj_pos, adj_neg, feat_table, W_bal, a_bal, W_unbal, a_unbal):
    h1f, h2f, scal = pl.pallas_call(
        _proj_body,
        grid=(NRB,),
        in_specs=[
            pl.BlockSpec((2, RB, D), lambda i: (0, i, 0)),
            pl.BlockSpec((D, D), lambda i: (0, 0)),
            pl.BlockSpec((2 * D, 1), lambda i: (0, 0)),
            pl.BlockSpec((D, D), lambda i: (0, 0)),
            pl.BlockSpec((2 * D, 1), lambda i: (0, 0)),
        ],
        out_specs=[
            pl.BlockSpec((NC, RB, D), lambda i: (0, i, 0)),
            pl.BlockSpec((NC, RB, D), lambda i: (0, i, 0)),
            pl.BlockSpec((NC, 4, NP), lambda i: (0, 0, 0)),
        ],
        out_shape=[
            jax.ShapeDtypeStruct((NC, NP, D), jnp.float32),
            jax.ShapeDtypeStruct((NC, NP, D), jnp.float32),
            jax.ShapeDtypeStruct((NC, 4, NP), jnp.float32),
        ],
    )(jnp.pad(feat_table, ((0, 0), (0, NP - N), (0, 0))),
      W_bal, a_bal, W_unbal, a_unbal)

    rows_self = jnp.arange(N, dtype=jnp.int32)
    e1 = _pad_chunked(
        jnp.concatenate([rows_self, adj_pos[0].astype(jnp.int32)]),
        jnp.concatenate([nodes.astype(jnp.int32), adj_pos[1].astype(jnp.int32)]),
        E1P)
    e2 = _pad_chunked(adj_neg[0], adj_neg[1], E2P)

    zrows = jnp.zeros((N, D), jnp.float32)

    acc, den = _edge_kernel(
        h1f.reshape(NC * NP, D), h2f.reshape(NC * NP, D), scal.reshape(-1),
        e1, e2, zrows)
    den = den.reshape(NC, N)

    x_bal, x_unbal = pl.pallas_call(
        _fin_body,
        out_shape=[
            jax.ShapeDtypeStruct((N, D), jnp.float32),
            jax.ShapeDtypeStruct((N, D), jnp.float32),
        ],
    )(acc, den)
    return (x_bal, x_unbal)
